# Initial kernel scaffold; baseline (speedup 1.0000x reference)
import jax
import jax.numpy as jnp
from jax.experimental import pallas as pl


def kernel(mapping, maps):
    raise NotImplementedError("write your pallas kernel here")


# speedup vs baseline: 1.3915x; 1.3915x over previous
import functools

import jax
import jax.numpy as jnp
from jax import lax
from jax.experimental import pallas as pl
from jax.experimental.pallas import tpu as pltpu
from jax.experimental.pallas import tpu_sc as plsc

DIM = 32
NUM_HEADS = 8
ROW = NUM_HEADS * DIM * DIM
CH = 4


def _sc_gather(table2d, idx2d):
    n_rows_out = idx2d.shape[0] * idx2d.shape[1]
    info = plsc.get_sparse_core_info()
    nc, ns = info.num_cores, info.num_subcores
    nw = nc * ns
    b_per_w = n_rows_out // nw
    n_chunks = b_per_w // CH

    mesh = plsc.VectorSubcoreMesh(core_axis_name="c", subcore_axis_name="s")

    @functools.partial(
        pl.kernel,
        mesh=mesh,
        out_type=jax.ShapeDtypeStruct((n_rows_out, ROW), jnp.float32),
        scratch_types=[
            pltpu.VMEM((n_chunks, CH), jnp.int32),
            pltpu.VMEM((CH, ROW), jnp.float32),
            pltpu.VMEM((CH, ROW), jnp.float32),
            pltpu.SemaphoreType.DMA,
            pltpu.SemaphoreType.DMA,
            pltpu.SemaphoreType.DMA,
            pltpu.SemaphoreType.DMA,
        ],
    )
    def k(table_hbm, idx_hbm, out_hbm, idx_v, buf0, buf1, g0, g1, s0, s1):
        wid = lax.axis_index("s") * nc + lax.axis_index("c")
        base = wid * b_per_w
        pltpu.sync_copy(idx_hbm.at[pl.ds(wid * n_chunks, n_chunks)], idx_v)

        bufs = (buf0, buf1)
        gsems = (g0, g1)
        ssems = (s0, s1)

        def gather_start(i, b):
            pltpu.make_async_copy(
                table_hbm.at[idx_v.at[i]], bufs[b], gsems[b]).start()

        def gather_wait(b):
            pltpu.make_async_copy(
                table_hbm.at[idx_v.at[0]], bufs[b], gsems[b]).wait()

        def scatter_start(i, b):
            pltpu.make_async_copy(
                bufs[b], out_hbm.at[pl.ds(base + i * CH, CH)], ssems[b]).start()

        def scatter_wait(b):
            pltpu.make_async_copy(
                bufs[b], out_hbm.at[pl.ds(base, CH)], ssems[b]).wait()

        gather_start(0, 0)
        gather_start(1, 1)
        gather_wait(0)
        scatter_start(0, 0)

        def loop_body(j, carry):
            i0 = 2 * j + 1
            scatter_wait(0)
            gather_start(i0 + 1, 0)
            gather_wait(1)
            scatter_start(i0, 1)
            i1 = 2 * j + 2
            scatter_wait(1)
            gather_start(i1 + 1, 1)
            gather_wait(0)
            scatter_start(i1, 0)
            return carry

        lax.fori_loop(0, (n_chunks - 2) // 2, loop_body, 0)

        scatter_wait(0)
        gather_wait(1)
        scatter_start(n_chunks - 1, 1)
        scatter_wait(1)

    return k


def kernel(mapping, maps):
    idx2d = jnp.ravel(mapping).astype(jnp.int32).reshape(-1, CH)
    table2d = maps.reshape(maps.shape[0], -1)
    out = _sc_gather(table2d, idx2d)(table2d, idx2d)
    return out.reshape(tuple(mapping.shape) + (NUM_HEADS, DIM, DIM))
